# baseline (device time: 99553 ns/iter reference)
import os

import jax
import jax.numpy as jnp
from jax import lax
from jax.experimental import pallas as pl
from jax.experimental.pallas import tpu as pltpu

P = 4
K = 16

_VARIANT = os.environ.get("DIAG_VARIANT", "")

PH_IN = 0
PH_MID = 1
PH_OUT = 2
PH_X = 3


def kernel(x):
    m, n = x.shape
    mh = m // 2
    rs = mh // K

    def body(x_ref, out_ref, r1_buf, r2_buf, send_sems, recv_sems):
        my_x = lax.axis_index("x")
        my_y = lax.axis_index("y")
        my_z = lax.axis_index("z")

        is_end = jnp.logical_or(my_y == 0, my_y == 3)
        inner_y = jnp.where(my_y == 0, 1, 2)
        end_y = jnp.where(my_y == 1, 0, 3)
        om_y = jnp.where(my_y == 1, 2, 1)
        xp = 1 - my_x

        barrier = pltpu.get_barrier_semaphore()

        @pl.when(is_end)
        def _():
            for dev in ((my_x, inner_y, my_z), (xp, my_y, my_z)):
                pl.semaphore_signal(
                    barrier, inc=1, device_id=dev,
                    device_id_type=pl.DeviceIdType.MESH,
                )
            pl.semaphore_wait(barrier, 2)

        @pl.when(jnp.logical_not(is_end))
        def _():
            for dev in (
                (my_x, end_y, my_z),
                (my_x, om_y, my_z),
                (xp, my_y, my_z),
            ):
                pl.semaphore_signal(
                    barrier, inc=1, device_id=dev,
                    device_id_type=pl.DeviceIdType.MESH,
                )
            pl.semaphore_wait(barrier, 3)

        def mk(phase, i, src, dst, dev):
            return pltpu.make_async_remote_copy(
                src_ref=src,
                dst_ref=dst,
                send_sem=send_sems.at[phase, i],
                recv_sem=recv_sems.at[phase, i],
                device_id=dev,
                device_id_type=pl.DeviceIdType.MESH,
            )

        r0 = my_x * mh
        o0 = mh - r0

        def end_program():
            s1 = []
            for i in range(K):
                g = pl.ds(r0 + i * rs, rs)
                l = pl.ds(i * rs, rs)
                out_ref[g, :] = x_ref[g, :].astype(out_ref.dtype)
                rdma = mk(PH_IN, i, out_ref.at[g], r1_buf.at[l],
                          (my_x, inner_y, my_z))
                rdma.start()
                s1.append(rdma)
            if _VARIANT == "ph1":
                for i in range(K):
                    s1[i].wait_send()
                return
            s4 = []
            for i in range(K):
                g = pl.ds(r0 + i * rs, rs)
                mk(PH_OUT, i, out_ref.at[g], out_ref.at[g],
                   (my_x, inner_y, my_z)).wait_recv()
                if _VARIANT != "nox":
                    rdma = mk(PH_X, i, out_ref.at[g], out_ref.at[g],
                              (xp, my_y, my_z))
                    rdma.start()
                    s4.append(rdma)
            if _VARIANT != "nox":
                for i in range(K):
                    og = pl.ds(o0 + i * rs, rs)
                    mk(PH_X, i, out_ref.at[og], out_ref.at[og],
                       (xp, my_y, my_z)).wait_recv()
            for i in range(K):
                s1[i].wait_send()
            for r in s4:
                r.wait_send()

        def mid_program():
            if _VARIANT == "ph1":
                for i in range(K):
                    l = pl.ds(i * rs, rs)
                    mk(PH_IN, i, r1_buf.at[l], r1_buf.at[l],
                       (my_x, end_y, my_z)).wait_recv()
                return
            s2, s3, s4 = [], [], []
            for i in range(K):
                g = pl.ds(r0 + i * rs, rs)
                l = pl.ds(i * rs, rs)
                mk(PH_IN, i, r1_buf.at[l], r1_buf.at[l],
                   (my_x, end_y, my_z)).wait_recv()
                if _VARIANT != "noadd":
                    out_ref[g, :] = (
                        x_ref[g, :].astype(out_ref.dtype) + r1_buf[l, :]
                    )
                rdma = mk(PH_MID, i, out_ref.at[g], r2_buf.at[l],
                          (my_x, om_y, my_z))
                rdma.start()
                s2.append(rdma)
                mk(PH_MID, i, out_ref.at[g], r2_buf.at[l],
                   (my_x, om_y, my_z)).wait_recv()
                s2[i].wait_send()
                if _VARIANT != "noadd":
                    out_ref[g, :] = out_ref[g, :] + r2_buf[l, :]
                rdma = mk(PH_OUT, i, out_ref.at[g], out_ref.at[g],
                          (my_x, end_y, my_z))
                rdma.start()
                s3.append(rdma)
                if _VARIANT != "nox":
                    rdma = mk(PH_X, i, out_ref.at[g], out_ref.at[g],
                              (xp, my_y, my_z))
                    rdma.start()
                    s4.append(rdma)
            if _VARIANT != "nox":
                for i in range(K):
                    og = pl.ds(o0 + i * rs, rs)
                    mk(PH_X, i, out_ref.at[og], out_ref.at[og],
                       (xp, my_y, my_z)).wait_recv()
            for i in range(K):
                s3[i].wait_send()
            for r in s4:
                r.wait_send()

        @pl.when(is_end)
        def _():
            end_program()

        @pl.when(jnp.logical_not(is_end))
        def _():
            mid_program()

    return pl.pallas_call(
        body,
        out_shape=jax.ShapeDtypeStruct((m, n), jnp.bfloat16),
        in_specs=[pl.BlockSpec(memory_space=pltpu.VMEM)],
        out_specs=pl.BlockSpec(memory_space=pltpu.VMEM),
        scratch_shapes=[
            pltpu.VMEM((m // 2, n), jnp.bfloat16),
            pltpu.VMEM((m // 2, n), jnp.bfloat16),
            pltpu.SemaphoreType.DMA((4, K)),
            pltpu.SemaphoreType.DMA((4, K)),
        ],
        compiler_params=pltpu.CompilerParams(collective_id=0),
    )(x)
